# Initial kernel scaffold; baseline (speedup 1.0000x reference)
import jax
import jax.numpy as jnp
from jax.experimental import pallas as pl


def kernel(x, W, b, binary_mask):
    raise NotImplementedError("write your pallas kernel here")


# speedup vs baseline: 2.2452x; 2.2452x over previous
import functools

import jax
import jax.numpy as jnp
from jax.experimental import pallas as pl
from jax.experimental.pallas import tpu as pltpu


def _masked_linear_kern(x_ref, w_ref, m_ref, b_ref, o_ref, wm_ref):
    @pl.when(pl.program_id(0) == 0)
    def _():
        wm_ref[...] = (w_ref[...] * m_ref[...]).astype(jnp.bfloat16)

    xb = x_ref[...].astype(jnp.bfloat16)
    acc = jax.lax.dot_general(
        xb, wm_ref[...],
        dimension_numbers=(((1,), (1,)), ((), ())),
        preferred_element_type=jnp.float32,
    )
    o_ref[...] = acc + b_ref[...]


@functools.partial(jax.jit, static_argnames=("bm",))
def _masked_linear(x2, W, b2, binary_mask, bm=1024):
    M, K = x2.shape
    N = W.shape[0]
    return pl.pallas_call(
        _masked_linear_kern,
        grid=(M // bm,),
        in_specs=[
            pl.BlockSpec((bm, K), lambda i: (i, 0)),
            pl.BlockSpec((N, K), lambda i: (0, 0)),
            pl.BlockSpec((N, K), lambda i: (0, 0)),
            pl.BlockSpec((1, N), lambda i: (0, 0)),
        ],
        out_specs=pl.BlockSpec((bm, N), lambda i: (i, 0)),
        out_shape=jax.ShapeDtypeStruct((M, N), jnp.float32),
        scratch_shapes=[pltpu.VMEM((N, K), jnp.bfloat16)],
    )(x2, W, binary_mask, b2)


def kernel(x, W, b, binary_mask):
    B, S, D = x.shape
    N = W.shape[0]
    out = _masked_linear(x.reshape(B * S, D), W, b.reshape(1, N), binary_mask)
    return out.reshape(B, S, N)


# speedup vs baseline: 2.2842x; 1.0174x over previous
import functools

import jax
import jax.numpy as jnp
from jax.experimental import pallas as pl
from jax.experimental.pallas import tpu as pltpu


def _masked_linear_kern(x_ref, w_ref, m_ref, b_ref, o_ref, wm_ref):
    @pl.when(pl.program_id(0) == 0)
    def _():
        wm_ref[...] = (w_ref[...] * m_ref[...]).astype(jnp.bfloat16)

    xb = x_ref[...].astype(jnp.bfloat16)
    acc = jax.lax.dot_general(
        xb, wm_ref[...],
        dimension_numbers=(((1,), (1,)), ((), ())),
        preferred_element_type=jnp.float32,
    )
    o_ref[...] = acc + b_ref[...]


@functools.partial(jax.jit, static_argnames=("bm",))
def _masked_linear(x2, W, b2, binary_mask, bm=2048):
    M, K = x2.shape
    N = W.shape[0]
    return pl.pallas_call(
        _masked_linear_kern,
        grid=(M // bm,),
        in_specs=[
            pl.BlockSpec((bm, K), lambda i: (i, 0)),
            pl.BlockSpec((N, K), lambda i: (0, 0)),
            pl.BlockSpec((N, K), lambda i: (0, 0)),
            pl.BlockSpec((1, N), lambda i: (0, 0)),
        ],
        out_specs=pl.BlockSpec((bm, N), lambda i: (i, 0)),
        out_shape=jax.ShapeDtypeStruct((M, N), jnp.float32),
        scratch_shapes=[pltpu.VMEM((N, K), jnp.bfloat16)],
    )(x2, W, binary_mask, b2)


def kernel(x, W, b, binary_mask):
    B, S, D = x.shape
    N = W.shape[0]
    out = _masked_linear(x.reshape(B * S, D), W, b.reshape(1, N), binary_mask)
    return out.reshape(B, S, N)
